# baseline (speedup 1.0000x reference)
import functools

import jax
import jax.numpy as jnp
from jax import lax
from jax.experimental import pallas as pl
from jax.experimental.pallas import tpu as pltpu
from jax.experimental.pallas import tpu_sc as plsc

N = 10000
NPAD = 10240
E = 320000
CH = 128
NW = 32
RPW = 79
EPW = RPW * CH
EPAD = NW * EPW
NTILES = 16
TPT = NPAD // NTILES
ZR = 64

SELU_ALPHA = 1.6732632423543772
SELU_SCALE = 1.0507009873554805


def _zero_vmem2d(ref, rows, cols):
    z = jnp.zeros((16,), jnp.float32)
    per_row = cols // 16

    def body(i, carry):
        r = i // per_row
        k = i % per_row
        ref[r, pl.ds(k * 16, 16)] = z
        return carry

    lax.fori_loop(0, rows * per_row, body, 0)


def _make_prop(D):
    mesh = plsc.VectorSubcoreMesh(core_axis_name="c", subcore_axis_name="s")

    @functools.partial(
        pl.kernel,
        out_type=jax.ShapeDtypeStruct((2, NPAD, D), jnp.float32),
        mesh=mesh,
        scratch_types=[
            pltpu.VMEM((RPW, CH), jnp.int32),
            pltpu.VMEM((RPW, CH), jnp.int32),
            pltpu.VMEM((CH, D), jnp.float32),
            pltpu.VMEM((ZR, D), jnp.float32),
            pltpu.VMEM_SHARED((NPAD, D), jnp.float32),
            pltpu.SemaphoreType.DMA,
        ],
        compiler_params=pltpu.CompilerParams(use_tc_tiling_on_sc=False),
    )
    def prop(h_hbm, src_hbm, dst_hbm, out_hbm, src_v, dst_v, rows_v, zb_v, acc,
             gsem):
        c = lax.axis_index("c")
        s = lax.axis_index("s")
        w = s * 2 + c
        _zero_vmem2d(zb_v, ZR, D)

        def zbody(z, carry):
            pltpu.sync_copy(zb_v, acc.at[pl.ds(s * TPT + z * ZR, ZR)])
            return carry

        lax.fori_loop(0, TPT // ZR, zbody, 0)
        pltpu.sync_copy(src_hbm.at[w], src_v)
        pltpu.sync_copy(dst_hbm.at[w], dst_v)
        plsc.subcore_barrier()

        def ebody(j, carry):
            pltpu.async_copy(h_hbm.at[src_v.at[j]], rows_v, gsem).wait()
            pltpu.sync_copy(rows_v, acc.at[dst_v.at[j]], add=True)
            return carry

        lax.fori_loop(0, RPW, ebody, 0)
        plsc.subcore_barrier()
        pltpu.sync_copy(acc.at[pl.ds(s * TPT, TPT)], out_hbm.at[c, pl.ds(s * TPT, TPT)])

    return prop


_sc_deg = _make_prop(16)
_sc_prop128 = _make_prop(128)
_sc_prop64 = _make_prop(64)
_sc_prop32 = _make_prop(32)


NB = 256
GRID = NPAD // NB


def _rowspec(d):
    return pl.BlockSpec((NB, d), lambda i: (i, 0))


def _fullspec(shape):
    nd = len(shape)
    return pl.BlockSpec(shape, lambda i: (0,) * nd)


def _tc_prep(degp, xp):
    def kern(degp_ref, x_ref, dinv_ref, dinv2_ref, dcinv_ref, s0_ref):
        deg = degp_ref[0][:, 0:1] + degp_ref[1][:, 0:1]
        dinv = jnp.where(deg > 0, lax.rsqrt(jnp.maximum(deg, 1e-12)), 0.0)
        dinv_ref[...] = dinv
        dinv2_ref[...] = dinv * dinv
        dcinv_ref[...] = 1.0 / jnp.maximum(deg, 1.0)
        s0_ref[...] = x_ref[...] * dinv

    return pl.pallas_call(
        kern,
        grid=(GRID,),
        in_specs=[
            pl.BlockSpec((2, NB, 16), lambda i: (0, i, 0)),
            _rowspec(128),
        ],
        out_specs=[_rowspec(1), _rowspec(1), _rowspec(1), _rowspec(128)],
        out_shape=[
            jax.ShapeDtypeStruct((NPAD, 1), jnp.float32),
            jax.ShapeDtypeStruct((NPAD, 1), jnp.float32),
            jax.ShapeDtypeStruct((NPAD, 1), jnp.float32),
            jax.ShapeDtypeStruct((NPAD, 128), jnp.float32),
        ],
    )(degp, xp)


def _tc_scale(ua, ub, sc):
    d = ua.shape[-1]

    def kern(a_ref, b_ref, s_ref, o_ref):
        o_ref[...] = (a_ref[...] + b_ref[...]) * s_ref[...]

    return pl.pallas_call(
        kern,
        grid=(GRID,),
        in_specs=[_rowspec(d), _rowspec(d), _rowspec(1)],
        out_specs=_rowspec(d),
        out_shape=jax.ShapeDtypeStruct((NPAD, d), jnp.float32),
    )(ua, ub, sc)


def _tc_conv1(v1a, v1b, dcinv, h, wl, wr, b):
    def kern(a_ref, b_ref, dc_ref, h_ref, wl_ref, wr_ref, bias_ref, o_ref):
        agg = (a_ref[...] + b_ref[...]) * dc_ref[...]
        z = (
            jnp.dot(agg, wl_ref[...], preferred_element_type=jnp.float32)
            + jnp.dot(h_ref[...], wr_ref[...], preferred_element_type=jnp.float32)
            + bias_ref[...]
        )
        o_ref[...] = SELU_SCALE * jnp.where(z > 0, z, SELU_ALPHA * (jnp.exp(z) - 1.0))

    return pl.pallas_call(
        kern,
        grid=(GRID,),
        in_specs=[
            _rowspec(128), _rowspec(128), _rowspec(1), _rowspec(128),
            _fullspec((128, 64)), _fullspec((128, 64)), _fullspec((1, 64)),
        ],
        out_specs=_rowspec(64),
        out_shape=jax.ShapeDtypeStruct((NPAD, 64), jnp.float32),
    )(v1a, v1b, dcinv, h, wl, wr, b)


def _tc_conv2(v2a, v2b, dcinv, h1, wl, wr, b, t, dinv):
    def kern(a_ref, b_ref, dc_ref, h_ref, wl_ref, wr_ref, bias_ref, t_ref,
             di_ref, p_ref, pyp_ref, s2_ref):
        agg = (a_ref[...] + b_ref[...]) * dc_ref[...]
        z = (
            jnp.dot(agg, wl_ref[...], preferred_element_type=jnp.float32)
            + jnp.dot(h_ref[...], wr_ref[...], preferred_element_type=jnp.float32)
            + bias_ref[...]
        )
        m = jnp.max(z, axis=1, keepdims=True)
        e = jnp.exp(z - m)
        p = e / jnp.sum(e, axis=1, keepdims=True)
        p_ref[...] = p
        pyp = jnp.dot(p, t_ref[...], preferred_element_type=jnp.float32)
        pyp_ref[...] = pyp
        s2_ref[...] = pyp * di_ref[...]

    return pl.pallas_call(
        kern,
        grid=(GRID,),
        in_specs=[
            _rowspec(64), _rowspec(64), _rowspec(1), _rowspec(64),
            _fullspec((64, 32)), _fullspec((64, 32)), _fullspec((1, 32)),
            _fullspec((32, 32)), _rowspec(1),
        ],
        out_specs=[_rowspec(32), _rowspec(32), _rowspec(32)],
        out_shape=[
            jax.ShapeDtypeStruct((NPAD, 32), jnp.float32),
            jax.ShapeDtypeStruct((NPAD, 32), jnp.float32),
            jax.ShapeDtypeStruct((NPAD, 32), jnp.float32),
        ],
    )(v2a, v2b, dcinv, h1, wl, wr, b, t, dinv)


def kernel(x, edge_index, T, Wl1, Wr1, b1, Wl2, Wr2, b2):
    src = edge_index[0].astype(jnp.int32)
    dst = edge_index[1].astype(jnp.int32)
    pad = EPAD - E
    src2d = jnp.concatenate([src, jnp.zeros((pad,), jnp.int32)]).reshape(NW, RPW, CH)
    dst_pad = N + (jnp.arange(pad, dtype=jnp.int32) % (NPAD - N))
    dst2d = jnp.concatenate([dst, dst_pad]).reshape(NW, RPW, CH)
    xp = jnp.pad(x, ((0, NPAD - N), (0, 0)))

    degp = _sc_deg(jnp.ones((NPAD, 16), jnp.float32), src2d, dst2d)
    dinv, dinv2, dcinv, s0 = _tc_prep(degp, xp)

    u1 = _sc_prop128(s0, src2d, dst2d)
    s1 = _tc_scale(u1[0], u1[1], dinv2)
    u2 = _sc_prop128(s1, src2d, dst2d)
    h = _tc_scale(u2[0], u2[1], dinv)

    v1 = _sc_prop128(h, src2d, dst2d)
    h1 = _tc_conv1(v1[0], v1[1], dcinv, h, Wl1.T, Wr1.T, b1.reshape(1, -1))

    v2 = _sc_prop64(h1, src2d, dst2d)
    p, pyp, s2 = _tc_conv2(
        v2[0], v2[1], dcinv, h1, Wl2.T, Wr2.T, b2.reshape(1, -1), T, dinv
    )

    w1 = _sc_prop32(s2, src2d, dst2d)
    s3 = _tc_scale(w1[0], w1[1], dinv2)
    w2 = _sc_prop32(s3, src2d, dst2d)
    pyt = _tc_scale(w2[0], w2[1], dinv)

    return (p[:N], pyp[:N], pyt[:N])


# speedup vs baseline: 1.1144x; 1.1144x over previous
import functools

import jax
import jax.numpy as jnp
from jax import lax
from jax.experimental import pallas as pl
from jax.experimental.pallas import tpu as pltpu
from jax.experimental.pallas import tpu_sc as plsc

N = 10000
NPAD = 10240
E = 320000
CH = 128
NW = 32
RPW = 79
EPW = RPW * CH
EPAD = NW * EPW
NTILES = 16
TPT = NPAD // NTILES
ZR = 64

SELU_ALPHA = 1.6732632423543772
SELU_SCALE = 1.0507009873554805


def _zero_vmem2d(ref, rows, cols):
    z = jnp.zeros((16,), jnp.float32)
    per_row = cols // 16

    def body(i, carry):
        r = i // per_row
        k = i % per_row
        ref[r, pl.ds(k * 16, 16)] = z
        return carry

    lax.fori_loop(0, rows * per_row, body, 0)


def _make_prop(D):
    mesh = plsc.VectorSubcoreMesh(core_axis_name="c", subcore_axis_name="s")
    segs = ((0, 40), (40, 39)) if D == 128 else ((0, RPW),)
    ib = max(l for _, l in segs)
    zr = 32 if D == 128 else 64

    @functools.partial(
        pl.kernel,
        out_type=jax.ShapeDtypeStruct((2, NPAD, D), jnp.float32),
        mesh=mesh,
        scratch_types=[
            pltpu.VMEM((ib, CH), jnp.int32),
            pltpu.VMEM((ib, CH), jnp.int32),
            pltpu.VMEM((2, CH, D), jnp.float32),
            pltpu.VMEM((zr, D), jnp.float32),
            pltpu.VMEM_SHARED((NPAD, D), jnp.float32),
            pltpu.SemaphoreType.DMA,
            pltpu.SemaphoreType.DMA,
        ],
        compiler_params=pltpu.CompilerParams(use_tc_tiling_on_sc=False),
    )
    def prop(h_hbm, src_hbm, dst_hbm, out_hbm, src_v, dst_v, rows_v, zb_v, acc,
             gsem, ssem):
        c = lax.axis_index("c")
        s = lax.axis_index("s")
        w = s * 2 + c
        _zero_vmem2d(zb_v, zr, D)

        def zbody(z, carry):
            pltpu.sync_copy(zb_v, acc.at[pl.ds(s * TPT + z * zr, zr)])
            return carry

        lax.fori_loop(0, TPT // zr, zbody, 0)
        plsc.subcore_barrier()

        for off, seg_len in segs:
            pltpu.sync_copy(src_hbm.at[w, pl.ds(off, seg_len)],
                            src_v.at[pl.ds(0, seg_len)])
            pltpu.sync_copy(dst_hbm.at[w, pl.ds(off, seg_len)],
                            dst_v.at[pl.ds(0, seg_len)])
            pltpu.async_copy(h_hbm.at[src_v.at[0]], rows_v.at[0], gsem)

            def ebody(j, carry, seg_len=seg_len):
                b = lax.rem(j, 2)
                nb = 1 - b
                pltpu.make_async_copy(
                    h_hbm.at[src_v.at[j]], rows_v.at[b], gsem
                ).wait()

                @pl.when(j > 0)
                def _():
                    pltpu.make_async_copy(
                        rows_v.at[nb], acc.at[dst_v.at[j - 1]], ssem
                    ).wait()

                @pl.when(j + 1 < seg_len)
                def _():
                    pltpu.async_copy(h_hbm.at[src_v.at[j + 1]], rows_v.at[nb], gsem)

                pltpu.async_copy(rows_v.at[b], acc.at[dst_v.at[j]], ssem, add=True)
                return carry

            lax.fori_loop(0, seg_len, ebody, 0)
            pltpu.make_async_copy(
                rows_v.at[(seg_len - 1) % 2], acc.at[dst_v.at[seg_len - 1]], ssem
            ).wait()
        plsc.subcore_barrier()
        pltpu.sync_copy(acc.at[pl.ds(s * TPT, TPT)], out_hbm.at[c, pl.ds(s * TPT, TPT)])

    return prop


_sc_deg = _make_prop(16)
_sc_prop128 = _make_prop(128)
_sc_prop64 = _make_prop(64)
_sc_prop32 = _make_prop(32)


NB = 256
GRID = NPAD // NB


def _rowspec(d):
    return pl.BlockSpec((NB, d), lambda i: (i, 0))


def _fullspec(shape):
    nd = len(shape)
    return pl.BlockSpec(shape, lambda i: (0,) * nd)


def _tc_prep(degp, xp):
    def kern(degp_ref, x_ref, dinv_ref, dinv2_ref, dcinv_ref, s0_ref):
        deg = degp_ref[0][:, 0:1] + degp_ref[1][:, 0:1]
        dinv = jnp.where(deg > 0, lax.rsqrt(jnp.maximum(deg, 1e-12)), 0.0)
        dinv_ref[...] = dinv
        dinv2_ref[...] = dinv * dinv
        dcinv_ref[...] = 1.0 / jnp.maximum(deg, 1.0)
        s0_ref[...] = x_ref[...] * dinv

    return pl.pallas_call(
        kern,
        grid=(GRID,),
        in_specs=[
            pl.BlockSpec((2, NB, 16), lambda i: (0, i, 0)),
            _rowspec(128),
        ],
        out_specs=[_rowspec(1), _rowspec(1), _rowspec(1), _rowspec(128)],
        out_shape=[
            jax.ShapeDtypeStruct((NPAD, 1), jnp.float32),
            jax.ShapeDtypeStruct((NPAD, 1), jnp.float32),
            jax.ShapeDtypeStruct((NPAD, 1), jnp.float32),
            jax.ShapeDtypeStruct((NPAD, 128), jnp.float32),
        ],
    )(degp, xp)


def _tc_scale(ua, ub, sc):
    d = ua.shape[-1]

    def kern(a_ref, b_ref, s_ref, o_ref):
        o_ref[...] = (a_ref[...] + b_ref[...]) * s_ref[...]

    return pl.pallas_call(
        kern,
        grid=(GRID,),
        in_specs=[_rowspec(d), _rowspec(d), _rowspec(1)],
        out_specs=_rowspec(d),
        out_shape=jax.ShapeDtypeStruct((NPAD, d), jnp.float32),
    )(ua, ub, sc)


def _tc_conv1(v1a, v1b, dcinv, h, wl, wr, b):
    def kern(a_ref, b_ref, dc_ref, h_ref, wl_ref, wr_ref, bias_ref, o_ref):
        agg = (a_ref[...] + b_ref[...]) * dc_ref[...]
        z = (
            jnp.dot(agg, wl_ref[...], preferred_element_type=jnp.float32)
            + jnp.dot(h_ref[...], wr_ref[...], preferred_element_type=jnp.float32)
            + bias_ref[...]
        )
        o_ref[...] = SELU_SCALE * jnp.where(z > 0, z, SELU_ALPHA * (jnp.exp(z) - 1.0))

    return pl.pallas_call(
        kern,
        grid=(GRID,),
        in_specs=[
            _rowspec(128), _rowspec(128), _rowspec(1), _rowspec(128),
            _fullspec((128, 64)), _fullspec((128, 64)), _fullspec((1, 64)),
        ],
        out_specs=_rowspec(64),
        out_shape=jax.ShapeDtypeStruct((NPAD, 64), jnp.float32),
    )(v1a, v1b, dcinv, h, wl, wr, b)


def _tc_conv2(v2a, v2b, dcinv, h1, wl, wr, b, t, dinv):
    def kern(a_ref, b_ref, dc_ref, h_ref, wl_ref, wr_ref, bias_ref, t_ref,
             di_ref, p_ref, pyp_ref, s2_ref):
        agg = (a_ref[...] + b_ref[...]) * dc_ref[...]
        z = (
            jnp.dot(agg, wl_ref[...], preferred_element_type=jnp.float32)
            + jnp.dot(h_ref[...], wr_ref[...], preferred_element_type=jnp.float32)
            + bias_ref[...]
        )
        m = jnp.max(z, axis=1, keepdims=True)
        e = jnp.exp(z - m)
        p = e / jnp.sum(e, axis=1, keepdims=True)
        p_ref[...] = p
        pyp = jnp.dot(p, t_ref[...], preferred_element_type=jnp.float32)
        pyp_ref[...] = pyp
        s2_ref[...] = pyp * di_ref[...]

    return pl.pallas_call(
        kern,
        grid=(GRID,),
        in_specs=[
            _rowspec(64), _rowspec(64), _rowspec(1), _rowspec(64),
            _fullspec((64, 32)), _fullspec((64, 32)), _fullspec((1, 32)),
            _fullspec((32, 32)), _rowspec(1),
        ],
        out_specs=[_rowspec(32), _rowspec(32), _rowspec(32)],
        out_shape=[
            jax.ShapeDtypeStruct((NPAD, 32), jnp.float32),
            jax.ShapeDtypeStruct((NPAD, 32), jnp.float32),
            jax.ShapeDtypeStruct((NPAD, 32), jnp.float32),
        ],
    )(v2a, v2b, dcinv, h1, wl, wr, b, t, dinv)


def kernel(x, edge_index, T, Wl1, Wr1, b1, Wl2, Wr2, b2):
    src = edge_index[0].astype(jnp.int32)
    dst = edge_index[1].astype(jnp.int32)
    pad = EPAD - E
    src2d = jnp.concatenate([src, jnp.zeros((pad,), jnp.int32)]).reshape(NW, RPW, CH)
    dst_pad = N + (jnp.arange(pad, dtype=jnp.int32) % (NPAD - N))
    dst2d = jnp.concatenate([dst, dst_pad]).reshape(NW, RPW, CH)
    xp = jnp.pad(x, ((0, NPAD - N), (0, 0)))

    degp = _sc_deg(jnp.ones((NPAD, 16), jnp.float32), src2d, dst2d)
    dinv, dinv2, dcinv, s0 = _tc_prep(degp, xp)

    u1 = _sc_prop128(s0, src2d, dst2d)
    s1 = _tc_scale(u1[0], u1[1], dinv2)
    u2 = _sc_prop128(s1, src2d, dst2d)
    h = _tc_scale(u2[0], u2[1], dinv)

    v1 = _sc_prop128(h, src2d, dst2d)
    h1 = _tc_conv1(v1[0], v1[1], dcinv, h, Wl1.T, Wr1.T, b1.reshape(1, -1))

    v2 = _sc_prop64(h1, src2d, dst2d)
    p, pyp, s2 = _tc_conv2(
        v2[0], v2[1], dcinv, h1, Wl2.T, Wr2.T, b2.reshape(1, -1), T, dinv
    )

    w1 = _sc_prop32(s2, src2d, dst2d)
    s3 = _tc_scale(w1[0], w1[1], dinv2)
    w2 = _sc_prop32(s3, src2d, dst2d)
    pyt = _tc_scale(w2[0], w2[1], dinv)

    return (p[:N], pyp[:N], pyt[:N])


# speedup vs baseline: 1.2517x; 1.1232x over previous
import functools

import jax
import jax.numpy as jnp
from jax import lax
from jax.experimental import pallas as pl
from jax.experimental.pallas import tpu as pltpu
from jax.experimental.pallas import tpu_sc as plsc

N = 10000
NPAD = 10240
E = 320000
CH = 128
NW = 32
RPW = 79
EPW = RPW * CH
EPAD = NW * EPW
NTILES = 16
TPT = NPAD // NTILES
ZR = 64

SELU_ALPHA = 1.6732632423543772
SELU_SCALE = 1.0507009873554805


def _zero_vmem2d(ref, rows, cols):
    z = jnp.zeros((16,), jnp.float32)
    per_row = cols // 16

    def body(i, carry):
        r = i // per_row
        k = i % per_row
        ref[r, pl.ds(k * 16, 16)] = z
        return carry

    lax.fori_loop(0, rows * per_row, body, 0)


def _make_prop(D):
    mesh = plsc.VectorSubcoreMesh(core_axis_name="c", subcore_axis_name="s")
    segs = ((0, 40), (40, 39)) if D == 128 else ((0, RPW),)
    ib = max(l for _, l in segs)
    zr = 32 if D == 128 else 64

    @functools.partial(
        pl.kernel,
        out_type=jax.ShapeDtypeStruct((2, NPAD, D), jnp.float32),
        mesh=mesh,
        scratch_types=[
            pltpu.VMEM((ib, CH), jnp.int32),
            pltpu.VMEM((ib, CH), jnp.int32),
            pltpu.VMEM((2, CH, D), jnp.float32),
            pltpu.VMEM((zr, D), jnp.float32),
            pltpu.VMEM_SHARED((NPAD, D), jnp.float32),
            pltpu.SemaphoreType.DMA,
            pltpu.SemaphoreType.DMA,
        ],
        compiler_params=pltpu.CompilerParams(use_tc_tiling_on_sc=False),
    )
    def prop(h_hbm, src_hbm, dst_hbm, out_hbm, src_v, dst_v, rows_v, zb_v, acc,
             gsem, ssem):
        c = lax.axis_index("c")
        s = lax.axis_index("s")
        w = s * 2 + c
        _zero_vmem2d(zb_v, zr, D)

        def zbody(z, carry):
            pltpu.sync_copy(zb_v, acc.at[pl.ds(s * TPT + z * zr, zr)])
            return carry

        lax.fori_loop(0, TPT // zr, zbody, 0)
        plsc.subcore_barrier()

        for off, seg_len in segs:
            pltpu.sync_copy(src_hbm.at[w, pl.ds(off, seg_len)],
                            src_v.at[pl.ds(0, seg_len)])
            pltpu.sync_copy(dst_hbm.at[w, pl.ds(off, seg_len)],
                            dst_v.at[pl.ds(0, seg_len)])
            pltpu.async_copy(h_hbm.at[src_v.at[0]], rows_v.at[0], gsem)

            def ebody(j, carry, seg_len=seg_len):
                b = lax.rem(j, 2)
                nb = 1 - b
                pltpu.make_async_copy(
                    h_hbm.at[src_v.at[j]], rows_v.at[b], gsem
                ).wait()

                @pl.when(j > 0)
                def _():
                    pltpu.make_async_copy(
                        rows_v.at[nb], acc.at[dst_v.at[j - 1]], ssem
                    ).wait()

                @pl.when(j + 1 < seg_len)
                def _():
                    pltpu.async_copy(h_hbm.at[src_v.at[j + 1]], rows_v.at[nb], gsem)

                pltpu.async_copy(rows_v.at[b], acc.at[dst_v.at[j]], ssem, add=True)
                return carry

            lax.fori_loop(0, seg_len, ebody, 0)
            pltpu.make_async_copy(
                rows_v.at[(seg_len - 1) % 2], acc.at[dst_v.at[seg_len - 1]], ssem
            ).wait()
        plsc.subcore_barrier()
        pltpu.sync_copy(acc.at[pl.ds(s * TPT, TPT)], out_hbm.at[c, pl.ds(s * TPT, TPT)])

    return prop


_sc_deg = _make_prop(16)
_sc_prop128 = _make_prop(128)
_sc_prop64 = _make_prop(64)
_sc_prop32 = _make_prop(32)


NB = 256
GRID = NPAD // NB


def _rowspec(d):
    return pl.BlockSpec((NB, d), lambda i: (i, 0))


def _fullspec(shape):
    nd = len(shape)
    return pl.BlockSpec(shape, lambda i: (0,) * nd)


def _tc_prep(degp, xp):
    def kern(degp_ref, x_ref, dinv_ref, dinv2_ref, dcinv_ref, s0_ref):
        deg = degp_ref[0][:, 0:1] + degp_ref[1][:, 0:1]
        dinv = jnp.where(deg > 0, lax.rsqrt(jnp.maximum(deg, 1e-12)), 0.0)
        dinv_ref[...] = dinv
        dinv2_ref[...] = dinv * dinv
        dcinv_ref[...] = 1.0 / jnp.maximum(deg, 1.0)
        s0_ref[...] = x_ref[...] * dinv

    return pl.pallas_call(
        kern,
        grid=(GRID,),
        in_specs=[
            pl.BlockSpec((2, NB, 16), lambda i: (0, i, 0)),
            _rowspec(128),
        ],
        out_specs=[_rowspec(1), _rowspec(1), _rowspec(1), _rowspec(128)],
        out_shape=[
            jax.ShapeDtypeStruct((NPAD, 1), jnp.float32),
            jax.ShapeDtypeStruct((NPAD, 1), jnp.float32),
            jax.ShapeDtypeStruct((NPAD, 1), jnp.float32),
            jax.ShapeDtypeStruct((NPAD, 128), jnp.float32),
        ],
    )(degp, xp)


def _tc_scale(ua, ub, sc):
    d = ua.shape[-1]

    def kern(a_ref, b_ref, s_ref, o_ref):
        o_ref[...] = (a_ref[...] + b_ref[...]) * s_ref[...]

    return pl.pallas_call(
        kern,
        grid=(GRID,),
        in_specs=[_rowspec(d), _rowspec(d), _rowspec(1)],
        out_specs=_rowspec(d),
        out_shape=jax.ShapeDtypeStruct((NPAD, d), jnp.float32),
    )(ua, ub, sc)


def _tc_conv1a(u2a, u2b, dinv, wl, wr):
    def kern(a_ref, b_ref, di_ref, wl_ref, wr_ref, hl_ref, hr_ref):
        h = (a_ref[...] + b_ref[...]) * di_ref[...]
        hl_ref[...] = jnp.dot(h, wl_ref[...], preferred_element_type=jnp.float32)
        hr_ref[...] = jnp.dot(h, wr_ref[...], preferred_element_type=jnp.float32)

    return pl.pallas_call(
        kern,
        grid=(GRID,),
        in_specs=[
            _rowspec(128), _rowspec(128), _rowspec(1),
            _fullspec((128, 64)), _fullspec((128, 64)),
        ],
        out_specs=[_rowspec(64), _rowspec(64)],
        out_shape=[
            jax.ShapeDtypeStruct((NPAD, 64), jnp.float32),
            jax.ShapeDtypeStruct((NPAD, 64), jnp.float32),
        ],
    )(u2a, u2b, dinv, wl, wr)


def _tc_conv1b(v1a, v1b, dcinv, hr, b, wl2, wr2):
    def kern(a_ref, b_ref, dc_ref, hr_ref, bias_ref, wl_ref, wr_ref,
             h1l_ref, h1r_ref):
        z = (a_ref[...] + b_ref[...]) * dc_ref[...] + hr_ref[...] + bias_ref[...]
        h1 = SELU_SCALE * jnp.where(z > 0, z, SELU_ALPHA * (jnp.exp(z) - 1.0))
        h1l_ref[...] = jnp.dot(h1, wl_ref[...], preferred_element_type=jnp.float32)
        h1r_ref[...] = jnp.dot(h1, wr_ref[...], preferred_element_type=jnp.float32)

    return pl.pallas_call(
        kern,
        grid=(GRID,),
        in_specs=[
            _rowspec(64), _rowspec(64), _rowspec(1), _rowspec(64),
            _fullspec((1, 64)), _fullspec((64, 32)), _fullspec((64, 32)),
        ],
        out_specs=[_rowspec(32), _rowspec(32)],
        out_shape=[
            jax.ShapeDtypeStruct((NPAD, 32), jnp.float32),
            jax.ShapeDtypeStruct((NPAD, 32), jnp.float32),
        ],
    )(v1a, v1b, dcinv, hr, b, wl2, wr2)


def _tc_conv2(v2a, v2b, dcinv, h1r, b, t, dinv):
    def kern(a_ref, b_ref, dc_ref, hr_ref, bias_ref, t_ref, di_ref,
             p_ref, pyp_ref, s2_ref):
        z = (a_ref[...] + b_ref[...]) * dc_ref[...] + hr_ref[...] + bias_ref[...]
        m = jnp.max(z, axis=1, keepdims=True)
        e = jnp.exp(z - m)
        p = e / jnp.sum(e, axis=1, keepdims=True)
        p_ref[...] = p
        pyp = jnp.dot(p, t_ref[...], preferred_element_type=jnp.float32)
        pyp_ref[...] = pyp
        s2_ref[...] = pyp * di_ref[...]

    return pl.pallas_call(
        kern,
        grid=(GRID,),
        in_specs=[
            _rowspec(32), _rowspec(32), _rowspec(1), _rowspec(32),
            _fullspec((1, 32)), _fullspec((32, 32)), _rowspec(1),
        ],
        out_specs=[_rowspec(32), _rowspec(32), _rowspec(32)],
        out_shape=[
            jax.ShapeDtypeStruct((NPAD, 32), jnp.float32),
            jax.ShapeDtypeStruct((NPAD, 32), jnp.float32),
            jax.ShapeDtypeStruct((NPAD, 32), jnp.float32),
        ],
    )(v2a, v2b, dcinv, h1r, b, t, dinv)


def kernel(x, edge_index, T, Wl1, Wr1, b1, Wl2, Wr2, b2):
    src = edge_index[0].astype(jnp.int32)
    dst = edge_index[1].astype(jnp.int32)
    pad = EPAD - E
    src2d = jnp.concatenate([src, jnp.zeros((pad,), jnp.int32)]).reshape(NW, RPW, CH)
    dst_pad = N + (jnp.arange(pad, dtype=jnp.int32) % (NPAD - N))
    dst2d = jnp.concatenate([dst, dst_pad]).reshape(NW, RPW, CH)
    xp = jnp.pad(x, ((0, NPAD - N), (0, 0)))

    degp = _sc_deg(jnp.ones((NPAD, 16), jnp.float32), src2d, dst2d)
    dinv, dinv2, dcinv, s0 = _tc_prep(degp, xp)

    u1 = _sc_prop128(s0, src2d, dst2d)
    s1 = _tc_scale(u1[0], u1[1], dinv2)
    u2 = _sc_prop128(s1, src2d, dst2d)

    hl, hr = _tc_conv1a(u2[0], u2[1], dinv, Wl1.T, Wr1.T)
    v1 = _sc_prop64(hl, src2d, dst2d)
    h1l, h1r = _tc_conv1b(
        v1[0], v1[1], dcinv, hr, b1.reshape(1, -1), Wl2.T, Wr2.T
    )

    v2 = _sc_prop32(h1l, src2d, dst2d)
    p, pyp, s2 = _tc_conv2(
        v2[0], v2[1], dcinv, h1r, b2.reshape(1, -1), T, dinv
    )

    w1 = _sc_prop32(s2, src2d, dst2d)
    s3 = _tc_scale(w1[0], w1[1], dinv2)
    w2 = _sc_prop32(s3, src2d, dst2d)
    pyt = _tc_scale(w2[0], w2[1], dinv)

    return (p[:N], pyp[:N], pyt[:N])
